# baseline (speedup 1.0000x reference)
import jax
import jax.numpy as jnp
from jax.experimental import pallas as pl
from jax.experimental.pallas import tpu as pltpu

SOM_H, SOM_W, D = 64, 64, 512
HW = SOM_H * SOM_W
BATCH = 4096
TB = 512
NT = BATCH // TB


def _som_kernel(x_ref, w_ref, coord_ref, idx_ref, wsq_ref, cr_ref, xs_ref):
    i = pl.program_id(0)

    @pl.when(i == 0)
    def _():
        w = w_ref[...]
        wsq_ref[...] = jnp.sum(w * w, axis=1)[None, :]

    par_w = jax.lax.rem(i, 2)
    base_w = par_w * TB
    base_r = (1 - par_w) * TB

    x = x_ref[...]
    xs_ref[pl.ds(base_w, TB), :] = jnp.sum(x * x, axis=1, keepdims=True)
    cr_ref[pl.ds(base_w, TB), :] = jax.lax.dot_general(
        x, w_ref[...], (((1,), (1,)), ((), ())),
        preferred_element_type=jnp.float32,
    )

    dist = (xs_ref[pl.ds(base_r, TB), :]
            - 2.0 * cr_ref[pl.ds(base_r, TB), :]) + wsq_ref[...]
    idx = jnp.argmin(dist, axis=1).astype(jnp.int32)
    idx_ref[...] = idx[:, None]
    coord_ref[...] = jnp.stack([idx // SOM_W, idx % SOM_W], axis=1)


def kernel(x, weights):
    wf = weights.reshape(HW, D)
    grid = (NT + 1,)
    coords, idx = pl.pallas_call(
        _som_kernel,
        grid=grid,
        in_specs=[
            pl.BlockSpec((TB, D), lambda i: (jnp.minimum(i, NT - 1), 0)),
            pl.BlockSpec((HW, D), lambda i: (0, 0)),
        ],
        out_specs=[
            pl.BlockSpec((TB, 2), lambda i: (jnp.maximum(i - 1, 0), 0)),
            pl.BlockSpec((TB, 1), lambda i: (jnp.maximum(i - 1, 0), 0)),
        ],
        out_shape=[
            jax.ShapeDtypeStruct((BATCH, 2), jnp.int32),
            jax.ShapeDtypeStruct((BATCH, 1), jnp.int32),
        ],
        scratch_shapes=[
            pltpu.VMEM((1, HW), jnp.float32),
            pltpu.VMEM((2 * TB, HW), jnp.float32),
            pltpu.VMEM((2 * TB, 1), jnp.float32),
        ],
    )(x, wf)
    return coords, idx[:, 0]


# speedup vs baseline: 1.4515x; 1.4515x over previous
import jax
import jax.numpy as jnp
from jax.experimental import pallas as pl
from jax.experimental.pallas import tpu as pltpu

SOM_H, SOM_W, D = 64, 64, 512
HW = SOM_H * SOM_W
BATCH = 4096
TB = 1024
NH = 2
HB = TB // NH


def _som_kernel(x_ref, w_ref, coord_ref, idx_ref, wsq_ref):
    @pl.when(pl.program_id(0) == 0)
    def _():
        w = w_ref[...]
        wsq_ref[...] = jnp.sum(w * w, axis=1)[None, :]

    for h in range(NH):
        sl = pl.ds(h * HB, HB)
        xh = x_ref[sl, :]
        x_sq = jnp.sum(xh * xh, axis=1, keepdims=True)
        cross = jax.lax.dot_general(
            xh, w_ref[...], (((1,), (1,)), ((), ())),
            preferred_element_type=jnp.float32,
        )
        dist = (x_sq - 2.0 * cross) + wsq_ref[...]
        idx = jnp.argmin(dist, axis=1).astype(jnp.int32)
        idx_ref[sl, :] = idx[:, None]
        coord_ref[sl, :] = jnp.stack([idx // SOM_W, idx % SOM_W], axis=1)


def kernel(x, weights):
    wf = weights.reshape(HW, D)
    grid = (BATCH // TB,)
    coords, idx = pl.pallas_call(
        _som_kernel,
        grid=grid,
        in_specs=[
            pl.BlockSpec((TB, D), lambda i: (i, 0)),
            pl.BlockSpec((HW, D), lambda i: (0, 0)),
        ],
        out_specs=[
            pl.BlockSpec((TB, 2), lambda i: (i, 0)),
            pl.BlockSpec((TB, 1), lambda i: (i, 0)),
        ],
        out_shape=[
            jax.ShapeDtypeStruct((BATCH, 2), jnp.int32),
            jax.ShapeDtypeStruct((BATCH, 1), jnp.int32),
        ],
        scratch_shapes=[pltpu.VMEM((1, HW), jnp.float32)],
    )(x, wf)
    return coords, idx[:, 0]
